# Initial kernel scaffold; baseline (speedup 1.0000x reference)
import jax
import jax.numpy as jnp
from jax.experimental import pallas as pl


def kernel(pos, node_input, node_attr, edge_index, params):
    raise NotImplementedError("write your pallas kernel here")


# speedup vs baseline: 1.9923x; 1.9923x over previous
import functools
import math

import jax
import jax.numpy as jnp
from jax import lax
from jax.experimental import pallas as pl
from jax.experimental.pallas import tpu as pltpu
from jax.experimental.pallas import tpu_sc as plsc

_N = 10000
_E = 320000
_DIMS = [128, 50, 50, 50, 128]
_NB = 10
_MAX_R = 3.5
_INV_SQRT_NN = 1.0 / math.sqrt(32.0)

_NC = 2
_NS = 16
_NW = _NC * _NS
_EPW = _E // _NW
_CH = 80
_NCH = _EPW // _CH
_NPS = _N // _NS
_ZR = 125

_SQ3 = math.sqrt(3.0)
_SQ15 = math.sqrt(15.0)
_SQ5H = math.sqrt(5.0) / 2.0
_YC = 1.14136 * math.exp(2.0)
_STEP = _MAX_R / (_NB + 1)

_mesh = plsc.VectorSubcoreMesh(core_axis_name="c", subcore_axis_name="s")
_sc_params = pltpu.CompilerParams(use_tc_tiling_on_sc=False)


def _edge_vec(pos_pad, src, dst):
  @functools.partial(
      pl.kernel,
      out_type=jax.ShapeDtypeStruct((_E, 16), jnp.float32),
      mesh=_mesh,
      compiler_params=_sc_params,
      scratch_types=[
          pltpu.VMEM((_CH,), jnp.int32),
          pltpu.VMEM((_CH,), jnp.int32),
          pltpu.VMEM((_CH, 16), jnp.float32),
          pltpu.VMEM((_CH, 16), jnp.float32),
          pltpu.SemaphoreType.DMA,
          pltpu.SemaphoreType.DMA,
      ],
  )
  def k(pos_hbm, src_hbm, dst_hbm, out_hbm, sidx, didx, arow, brow, sem1, sem2):
    c = lax.axis_index("c")
    s = lax.axis_index("s")
    ebase = (c * _NS + s) * _EPW

    @pl.loop(0, _NCH)
    def _(t):
      off = ebase + t * _CH
      pltpu.sync_copy(src_hbm.at[pl.ds(off, _CH)], sidx)
      pltpu.sync_copy(dst_hbm.at[pl.ds(off, _CH)], didx)
      cp1 = pltpu.async_copy(pos_hbm.at[sidx], arow, sem1)
      cp2 = pltpu.async_copy(pos_hbm.at[didx], brow, sem2)
      cp1.wait()
      cp2.wait()

      @pl.loop(0, _CH)
      def _(e):
        arow[e, :] = arow[e, :] - brow[e, :]

      pltpu.sync_copy(arow, out_hbm.at[pl.ds(off, _CH)])

  return k(pos_pad, src, dst)


def _sc_agg(xin_pad, wes, src, dst, dp):
  @functools.partial(
      pl.kernel,
      out_type=jax.ShapeDtypeStruct((_NC, _N, dp), jnp.float32),
      mesh=_mesh,
      compiler_params=_sc_params,
      scratch_types=[
          pltpu.VMEM((_CH,), jnp.int32),
          pltpu.VMEM((_CH,), jnp.int32),
          pltpu.VMEM((_CH, dp), jnp.float32),
          pltpu.VMEM((_CH, dp), jnp.float32),
          pltpu.VMEM((_ZR, dp), jnp.float32),
          pltpu.VMEM_SHARED((_N, dp), jnp.float32),
          pltpu.SemaphoreType.DMA,
      ],
  )
  def k(xin_hbm, wes_hbm, src_hbm, dst_hbm, out_hbm,
        sidx, didx, rows, wrows, zbuf, acc, sem):
    c = lax.axis_index("c")
    s = lax.axis_index("s")

    @pl.loop(0, _ZR)
    def _(i):
      for j in range(dp // 16):
        zbuf[i, pl.ds(j * 16, 16)] = jnp.zeros((16,), jnp.float32)

    @pl.loop(0, _NPS // _ZR)
    def _(i):
      pltpu.sync_copy(zbuf, acc.at[pl.ds(s * _NPS + i * _ZR, _ZR)])

    plsc.subcore_barrier()

    ebase = (c * _NS + s) * _EPW

    @pl.loop(0, _NCH)
    def _(t):
      off = ebase + t * _CH
      pltpu.sync_copy(src_hbm.at[pl.ds(off, _CH)], sidx)
      pltpu.sync_copy(dst_hbm.at[pl.ds(off, _CH)], didx)
      pltpu.async_copy(xin_hbm.at[sidx], rows, sem).wait()
      pltpu.sync_copy(wes_hbm.at[pl.ds(off, _CH)], wrows)

      @pl.loop(0, _CH)
      def _(e):
        for j in range(dp // 16):
          sl = pl.ds(j * 16, 16)
          rows[e, sl] = rows[e, sl] * wrows[e, sl]

      pltpu.sync_copy(rows, acc.at[didx], add=True)

    plsc.subcore_barrier()

    @pl.loop(0, _NPS // _ZR)
    def _(i):
      r0 = s * _NPS + i * _ZR
      pltpu.sync_copy(acc.at[pl.ds(r0, _ZR)], out_hbm.at[c, pl.ds(r0, _ZR)])

  return k(xin_pad, wes, src, dst)


_BE = 4000


def _edge_coeffs_body(ev_ref, *refs):
  w1 = refs[0:4]
  b1 = refs[4:8]
  w2 = refs[8:12]
  p = refs[12:16]
  outs = refs[16:20]

  x = ev_ref[:, 0:1]
  y = ev_ref[:, 1:2]
  z = ev_ref[:, 2:3]
  r = jnp.sqrt(x * x + y * y + z * z)
  rs = jnp.maximum(r, 1e-9)
  ux = x / rs
  uy = y / rs
  uz = z / rs

  centers = ((lax.broadcasted_iota(jnp.int32, (1, _NB), 1)
              .astype(jnp.float32) + 1.0) * _STEP)
  diff = (r - centers) / _STEP
  inside = jnp.abs(diff) < 1.0
  d = jnp.where(inside, diff, 0.0)
  emb = jnp.where(inside, _YC * jnp.exp(1.0 / (d * d - 1.0)), 0.0)
  emb = emb * math.sqrt(float(_NB))

  for i in range(4):
    pv = p[i]
    es = (pv[0, 0]
          + pv[0, 1] * (_SQ3 * ux) + pv[0, 2] * (_SQ3 * uy)
          + pv[0, 3] * (_SQ3 * uz)
          + pv[0, 4] * (_SQ15 * ux * uy) + pv[0, 5] * (_SQ15 * uy * uz)
          + pv[0, 6] * (_SQ5H * (3.0 * uz * uz - 1.0))
          + pv[0, 7] * (_SQ15 * ux * uz)
          + pv[0, 8] * ((_SQ15 / 2.0) * (ux * ux - uy * uy)))
    h = jnp.maximum(
        jnp.dot(emb, w1[i][...], preferred_element_type=jnp.float32)
        + b1[i][...], 0.0)
    w = jnp.dot(h, w2[i][...], preferred_element_type=jnp.float32)
    wes = w * es
    din = _DIMS[i]
    if i == 0:
      outs[i][...] = wes
    else:
      outs[i][:, :din] = wes
      outs[i][:, din:] = jnp.zeros((_BE, 64 - din), jnp.float32)


def _edge_coeffs(evec, params):
  nblk = _E // _BE
  wshape = lambda a: pl.BlockSpec(a.shape, lambda i: (0,) * a.ndim)
  w1 = [params[f"l{i}_W1"] for i in range(4)]
  b1 = [params[f"l{i}_b1"].reshape(1, -1) for i in range(4)]
  w2 = [params[f"l{i}_W2"] for i in range(4)]
  p = [params[f"l{i}_p"].reshape(1, -1) for i in range(4)]
  in_specs = ([pl.BlockSpec((_BE, 16), lambda i: (i, 0))]
              + [wshape(a) for a in (w1 + b1 + w2 + p)])
  out_dims = [128, 64, 64, 64]
  out_specs = [pl.BlockSpec((_BE, dpo), lambda i: (i, 0)) for dpo in out_dims]
  out_shape = [jax.ShapeDtypeStruct((_E, dpo), jnp.float32) for dpo in out_dims]
  return pl.pallas_call(
      _edge_coeffs_body,
      grid=(nblk,),
      in_specs=in_specs,
      out_specs=out_specs,
      out_shape=out_shape,
  )(evec, *w1, *b1, *w2, *p)


def _prep(node_input, node_attr, pa):
  def body(ni_ref, na_ref, pa_ref, o_ref):
    o_ref[...] = ni_ref[...] * (na_ref[...] * pa_ref[0, 0])

  return pl.pallas_call(
      body,
      out_shape=jax.ShapeDtypeStruct((_N, 128), jnp.float32),
  )(node_input, node_attr, pa)


def _update(agg2, xin, wmsg, wskip, node_attr, pa_next, layer):
  din = _DIMS[layer]
  dout = _DIMS[layer + 1]
  last = layer == 3
  dpo = 128 if last else 64

  def body(agg_ref, xin_ref, wm_ref, ws_ref, na_ref, pa_ref, o_ref):
    a = (agg_ref[0] + agg_ref[1]) * _INV_SQRT_NN
    a = a[:, :din]
    xi = xin_ref[:, :din]
    yv = (jnp.dot(a, wm_ref[...], preferred_element_type=jnp.float32)
          + jnp.dot(xi, ws_ref[...], preferred_element_type=jnp.float32))
    if last:
      o_ref[...] = yv
    else:
      yv = jax.nn.gelu(yv) * (na_ref[...] * pa_ref[0, 0])
      o_ref[:, :dout] = yv
      o_ref[:, dout:] = jnp.zeros((_N, dpo - dout), jnp.float32)

  return pl.pallas_call(
      body,
      out_shape=jax.ShapeDtypeStruct((_N, dpo), jnp.float32),
  )(agg2, xin, wmsg, wskip, node_attr, pa_next)


def kernel(pos, node_input, node_attr, edge_index, params):
  src = edge_index[0].astype(jnp.int32)
  dst = edge_index[1].astype(jnp.int32)
  pos_pad = jnp.pad(pos, ((0, 0), (0, 13)))
  evec = _edge_vec(pos_pad, src, dst)
  wes = _edge_coeffs(evec, params)
  xin = _prep(node_input, node_attr, params["l0_pa"])
  for i in range(4):
    dp = 128 if i == 0 else 64
    agg2 = _sc_agg(xin, wes[i], src, dst, dp)
    pa_next = params[f"l{i + 1}_pa"] if i < 3 else params["l0_pa"]
    xin = _update(agg2, xin, params[f"l{i}_Wmsg"], params[f"l{i}_Wskip"],
                  node_attr, pa_next, i)
  return xin
